# baseline (speedup 1.0000x reference)
import functools

import jax
from jax import lax
from jax.experimental import pallas as pl
from jax.experimental.pallas import tpu as pltpu
from jax.experimental.pallas import tpu_sc as plsc


def _chunk_sizes(rows_per_w):
    if rows_per_w == 256:
        return [16, 64, 56, 64, 56]
    first = min(8, rows_per_w)
    rest = rows_per_w - first
    sizes = [first]
    while rest > 0:
        c = min(56, rest)
        sizes.append(c)
        rest -= c
    return sizes


def _broadcast_rows(table, batch):
    S, D = table.shape
    info = plsc.get_sparse_core_info()
    nw = info.num_cores * info.num_subcores
    rows_per_w = S // nw
    chunks = _chunk_sizes(rows_per_w)
    starts = [sum(chunks[:i]) for i in range(len(chunks))]
    buf0_rows = max(chunks[0::2])
    buf1_rows = max(chunks[1::2]) if len(chunks) > 1 else 1
    mesh = plsc.VectorSubcoreMesh(core_axis_name="c", subcore_axis_name="s")

    @functools.partial(
        pl.kernel,
        mesh=mesh,
        out_type=jax.ShapeDtypeStruct((batch * S, D), table.dtype),
        scratch_types=[
            pltpu.VMEM((buf0_rows, D), table.dtype),
            pltpu.VMEM((buf1_rows, D), table.dtype),
            pltpu.SemaphoreType.DMA,
            pltpu.SemaphoreType.DMA,
        ],
    )
    def k(table_hbm, out_hbm, buf0, buf1, rsem0, rsem1):
        wid = lax.axis_index("s") * info.num_cores + lax.axis_index("c")
        base = wid * rows_per_w
        bufs, rsems = (buf0, buf1), (rsem0, rsem1)

        def start_read(c):
            rows, slot = chunks[c], c % 2
            return pltpu.async_copy(
                table_hbm.at[pl.ds(base + starts[c], rows), :],
                bufs[slot].at[pl.ds(0, rows), :],
                rsems[slot])

        reads = {0: start_read(0)}
        if len(chunks) > 1:
            reads[1] = start_read(1)
        for c in range(len(chunks)):
            rows, slot = chunks[c], c % 2
            reads.pop(c).wait()
            for b in range(batch):
                pltpu.sync_copy(
                    bufs[slot].at[pl.ds(0, rows), :],
                    out_hbm.at[pl.ds(b * S + base + starts[c], rows), :])
            if c + 2 < len(chunks):
                reads[c + 2] = start_read(c + 2)

    return k(table)


def kernel(x, pos_embedding):
    batch, seq = x.shape
    table = pos_embedding[:seq]
    out = _broadcast_rows(table, batch)
    return out.reshape(batch, seq, pos_embedding.shape[1])
